# baseline (speedup 1.0000x reference)
import numpy as np
import jax
import jax.numpy as jnp
from jax.experimental import pallas as pl
from jax.experimental.pallas import tpu as pltpu

_P = 0.5
_MAX_DROP = 8


def _drop_indices():
    rng = np.random.RandomState(1)
    if not (rng.rand() < _P):
        return np.zeros((0,), np.int32)
    num_drop = int(rng.randint(1, _MAX_DROP + 1))
    return np.sort(rng.permutation(96)[:num_drop].astype(np.int32))


_DROP = tuple(int(i) for i in _drop_indices())

_B, _C, _H, _W = 16, 96, 224, 224

_RUNS = []
_prev = 0
for _d in _DROP:
    if _d > _prev:
        _RUNS.append((_prev, _d - _prev))
    _prev = _d + 1
if _prev < _C:
    _RUNS.append((_prev, _C - _prev))


def _body(in_hbm, out_hbm, zero_vmem, sem):
    zero_vmem[...] = jnp.zeros((1, 1, _H, _W), jnp.float32)
    copies = []
    for b in range(_B):
        for c0, ln in _RUNS:
            cp = pltpu.make_async_copy(
                in_hbm.at[pl.ds(b, 1), pl.ds(c0, ln)],
                out_hbm.at[pl.ds(b, 1), pl.ds(c0, ln)],
                sem,
            )
            cp.start()
            copies.append(cp)
        for d in _DROP:
            cp = pltpu.make_async_copy(
                zero_vmem,
                out_hbm.at[pl.ds(b, 1), pl.ds(d, 1)],
                sem,
            )
            cp.start()
            copies.append(cp)
    for cp in copies:
        cp.wait()


def kernel(image):
    return pl.pallas_call(
        _body,
        in_specs=[pl.BlockSpec(memory_space=pl.ANY)],
        out_specs=pl.BlockSpec(memory_space=pl.ANY),
        out_shape=jax.ShapeDtypeStruct((_B, _C, _H, _W), jnp.float32),
        scratch_shapes=[
            pltpu.VMEM((1, 1, _H, _W), jnp.float32),
            pltpu.SemaphoreType.DMA,
        ],
    )(image)


# speedup vs baseline: 47.4782x; 47.4782x over previous
import numpy as np
import jax
import jax.numpy as jnp
from jax.experimental import pallas as pl
from jax.experimental.pallas import tpu as pltpu

_P = 0.5
_MAX_DROP = 8


def _drop_indices():
    rng = np.random.RandomState(1)
    if not (rng.rand() < _P):
        return np.zeros((0,), np.int32)
    num_drop = int(rng.randint(1, _MAX_DROP + 1))
    return np.sort(rng.permutation(96)[:num_drop].astype(np.int32))


_DROP = tuple(int(i) for i in _drop_indices())

_B, _C, _H, _W = 16, 96, 224, 224

_RUNS = []
_prev = 0
for _d in _DROP:
    if _d > _prev:
        _RUNS.append((_prev, _d - _prev))
    _prev = _d + 1
if _prev < _C:
    _RUNS.append((_prev, _C - _prev))


def _body(in_hbm, out_hbm, buf, in_sems, out_sems):
    for i in range(2):
        for d in _DROP:
            buf[i, d] = jnp.zeros((_H, _W), jnp.float32)

    def start_in(b):
        i = b % 2
        cps = []
        for c0, ln in _RUNS:
            cp = pltpu.make_async_copy(
                in_hbm.at[b, pl.ds(c0, ln)],
                buf.at[i, pl.ds(c0, ln)],
                in_sems.at[i],
            )
            cp.start()
            cps.append(cp)
        return cps

    def start_out(b):
        i = b % 2
        cp = pltpu.make_async_copy(buf.at[i], out_hbm.at[b], out_sems.at[i])
        cp.start()
        return cp

    copies_in = {0: start_in(0)}
    copies_out = {}
    for b in range(_B):
        if b + 1 < _B:
            if b >= 1:
                copies_out[b - 1].wait()
            copies_in[b + 1] = start_in(b + 1)
        for cp in copies_in[b]:
            cp.wait()
        copies_out[b] = start_out(b)
    copies_out[_B - 2].wait()
    copies_out[_B - 1].wait()


def kernel(image):
    return pl.pallas_call(
        _body,
        in_specs=[pl.BlockSpec(memory_space=pl.ANY)],
        out_specs=pl.BlockSpec(memory_space=pl.ANY),
        out_shape=jax.ShapeDtypeStruct((_B, _C, _H, _W), jnp.float32),
        scratch_shapes=[
            pltpu.VMEM((2, _C, _H, _W), jnp.float32),
            pltpu.SemaphoreType.DMA((2,)),
            pltpu.SemaphoreType.DMA((2,)),
        ],
    )(image)
